# Initial kernel scaffold; baseline (speedup 1.0000x reference)
import jax
import jax.numpy as jnp
from jax.experimental import pallas as pl


def kernel(node_attr, edge_index, edge_attr, lin1_W, lin1_b, gate_lin1_W, gate_att_l, gate_att_r, gate_lin2_W, gate_bias, gru0_Wih, gru0_Whh, gru0_bih, gru0_bhh, atom_lin_W, atom_att_src, atom_att_dst, atom_bias, atom_gru_Wih, atom_gru_Whh, atom_gru_bih, atom_gru_bhh, mol_lin_W, mol_att_src, mol_att_dst, mol_bias, mol_gru_Wih, mol_gru_Whh, mol_gru_bih, mol_gru_bhh, lin2_W, lin2_b):
    raise NotImplementedError("write your pallas kernel here")


# speedup vs baseline: 15.8632x; 15.8632x over previous
import functools

import jax
import jax.numpy as jnp
from jax import lax
from jax.experimental import pallas as pl
from jax.experimental.pallas import tpu as pltpu
from jax.experimental.pallas import tpu_sc as plsc

N = 10000
E = 320000
H = 128
L = 5
NC = 2
NS = 16
NT = NC * NS
EPT = E // NT + (-(E // NT) % 64)
E_PAD = NT * EPT
ROWW = 144
NP = 10112
RPT = NP // NS

CA = 64
NCHA = EPT // CA
CG = 32
NCHG = EPT // CG

_f32 = jnp.float32
_i32 = jnp.int32


def _leaky(x):
    return jnp.where(x >= 0, x, 0.01 * x)


def _elu(x):
    return jnp.where(x > 0, x, jnp.exp(x) - 1.0)


def _zero_rows(buf, rows):
    zf = jnp.full((16,), 0.0, _f32)

    def _zrow(i, c2):
        for cc in range(ROWW // 16):
            buf[i, pl.ds(cc * 16, 16)] = zf
        return c2
    lax.fori_loop(0, rows, _zrow, 0)


def _zero_acc_slice(sbuf, accs, base, c):
    nfull = RPT // c
    for k in range(nfull):
        pltpu.sync_copy(sbuf, accs.at[pl.ds(base + k * c, c)])
    rem = RPT - nfull * c
    if rem:
        pltpu.sync_copy(sbuf.at[pl.ds(0, rem)],
                        accs.at[pl.ds(base + nfull * c, rem)])


def _sc_gat_body(xle_h, d_h, src_h, dst_h, acc_out,
                 srcv, dstv, gbuf, dbuf, sbuf, accs, gsem, dsem, ssem):
    cid = lax.axis_index("c")
    sid = lax.axis_index("s")
    wid = sid * NC + cid
    lane = lax.iota(_i32, 16)
    zeros16 = jnp.full((16,), 0, _i32)
    colH16 = jnp.full((16,), H, _i32)

    pltpu.sync_copy(src_h.at[wid], srcv)
    pltpu.sync_copy(dst_h.at[wid], dstv)

    _zero_rows(sbuf, CA)
    base = sid * RPT
    _zero_acc_slice(sbuf, accs, base, CA)
    plsc.subcore_barrier()

    def _chunk(g, carry):
        cp1 = pltpu.async_copy(xle_h.at[srcv.at[g]], gbuf, gsem)
        cp2 = pltpu.async_copy(d_h.at[dstv.at[g]], dbuf, dsem)
        cp1.wait()
        cp2.wait()

        def _grp(j, c2):
            idx16 = j * 16 + lane
            s16 = plsc.load_gather(gbuf, [idx16, colH16])
            d16 = plsc.load_gather(dbuf, [idx16, zeros16])
            a = _leaky(s16 + d16)
            eid = wid * EPT + g * CA + idx16
            ex16 = jnp.where(eid < E, jnp.exp(a), 0.0)
            for k in range(16):
                i = j * 16 + k
                ex = ex16[k]
                for cc in range(H // 16):
                    sbuf[i, pl.ds(cc * 16, 16)] = (
                        gbuf[i, pl.ds(cc * 16, 16)] * ex)
                sbuf[i, pl.ds(H, 16)] = jnp.where(lane == 0, ex, 0.0)
            return c2
        lax.fori_loop(0, CA // 16, _grp, 0)

        pltpu.async_copy(sbuf, accs.at[dstv.at[g]], ssem, add=True).wait()
        return carry
    lax.fori_loop(0, NCHA, _chunk, 0)

    plsc.subcore_barrier()
    pltpu.sync_copy(accs.at[pl.ds(base, RPT)],
                    acc_out.at[cid, pl.ds(base, RPT)])


@functools.lru_cache(maxsize=None)
def _sc_gat_kernel():
    return pl.kernel(
        _sc_gat_body,
        out_type=jax.ShapeDtypeStruct((NC, NP, ROWW), _f32),
        mesh=plsc.VectorSubcoreMesh(core_axis_name="c", subcore_axis_name="s"),
        compiler_params=pltpu.CompilerParams(use_tc_tiling_on_sc=False,
                                             needs_layout_passes=False),
        scratch_types=[
            pltpu.VMEM((NCHA, CA), _i32),
            pltpu.VMEM((NCHA, CA), _i32),
            pltpu.VMEM((CA, ROWW), _f32),
            pltpu.VMEM((CA, 8), _f32),
            pltpu.VMEM((CA, ROWW), _f32),
            pltpu.VMEM_SHARED((NP, ROWW), _f32),
            pltpu.SemaphoreType.DMA,
            pltpu.SemaphoreType.DMA,
            pltpu.SemaphoreType.DMA,
        ],
    )


def _sc_gat(*args):
    return _sc_gat_kernel()(*args)


def _sc_gate_body(uy_h, r_h, wrow_h, attl_h, src_h, dst_h, ea_h, acc_out,
                  srcv, dstv, eav, wv, lv, gbuf, rbuf, sbuf, accs,
                  gsem, rsem, ssem):
    cid = lax.axis_index("c")
    sid = lax.axis_index("s")
    wid = sid * NC + cid
    lane = lax.iota(_i32, 16)
    zeros16 = jnp.full((16,), 0, _i32)

    pltpu.sync_copy(src_h.at[wid], srcv)
    pltpu.sync_copy(dst_h.at[wid], dstv)
    pltpu.sync_copy(wrow_h, wv)
    pltpu.sync_copy(attl_h, lv)

    _zero_rows(sbuf, CG)
    base = sid * RPT
    _zero_acc_slice(sbuf, accs, base, CG)
    plsc.subcore_barrier()

    def _chunk(g, carry):
        cp1 = pltpu.async_copy(uy_h.at[srcv.at[g]], gbuf, gsem)
        cp2 = pltpu.async_copy(r_h.at[dstv.at[g]], rbuf, rsem)
        pltpu.sync_copy(ea_h.at[wid, g], eav)
        cp1.wait()
        cp2.wait()

        def _grp(j, c2):
            idx16 = j * 16 + lane
            r16 = plsc.load_gather(rbuf, [idx16, zeros16])
            ea16 = eav[pl.ds(j * 16, 16)]
            eid = wid * EPT + g * CG + idx16
            valid16 = jnp.where(eid < E, 1.0, 0.0)
            tv = jnp.full((16,), 0.0, _f32)
            for k in range(16):
                i = j * 16 + k
                e = ea16[k]
                t16 = jnp.full((16,), 0.0, _f32)
                for cc in range(H // 16):
                    z = (gbuf[i, pl.ds(cc * 16, 16)]
                         + e * wv[pl.ds(cc * 16, 16)])
                    t16 = t16 + _leaky(z) * lv[pl.ds(cc * 16, 16)]
                tv = jnp.where(lane == k, jnp.sum(t16), tv)
            ex16 = jnp.exp(_leaky(tv + r16)) * valid16
            for k in range(16):
                i = j * 16 + k
                ex = ex16[k]
                for cc in range(H // 16):
                    sbuf[i, pl.ds(cc * 16, 16)] = (
                        gbuf[i, pl.ds(H + cc * 16, 16)] * ex)
                sbuf[i, pl.ds(H, 16)] = jnp.where(lane == 0, ex, 0.0)
            return c2
        lax.fori_loop(0, CG // 16, _grp, 0)

        pltpu.async_copy(sbuf, accs.at[dstv.at[g]], ssem, add=True).wait()
        return carry
    lax.fori_loop(0, NCHG, _chunk, 0)

    plsc.subcore_barrier()
    pltpu.sync_copy(accs.at[pl.ds(base, RPT)],
                    acc_out.at[cid, pl.ds(base, RPT)])


@functools.lru_cache(maxsize=None)
def _sc_gate_kernel():
    return pl.kernel(
        _sc_gate_body,
        out_type=jax.ShapeDtypeStruct((NC, NP, ROWW), _f32),
        mesh=plsc.VectorSubcoreMesh(core_axis_name="c", subcore_axis_name="s"),
        compiler_params=pltpu.CompilerParams(use_tc_tiling_on_sc=False,
                                             needs_layout_passes=False),
        scratch_types=[
            pltpu.VMEM((NCHG, CG), _i32),
            pltpu.VMEM((NCHG, CG), _i32),
            pltpu.VMEM((CG,), _f32),
            pltpu.VMEM((H,), _f32),
            pltpu.VMEM((H,), _f32),
            pltpu.VMEM((CG, 2 * H), _f32),
            pltpu.VMEM((CG, 8), _f32),
            pltpu.VMEM((CG, ROWW), _f32),
            pltpu.VMEM_SHARED((NP, ROWW), _f32),
            pltpu.SemaphoreType.DMA,
            pltpu.SemaphoreType.DMA,
            pltpu.SemaphoreType.DMA,
        ],
    )


def _sc_gate(*args):
    return _sc_gate_kernel()(*args)


def _dot(a, b):
    return jax.lax.dot_general(a, b, (((1,), (0,)), ((), ())),
                               preferred_element_type=_f32)


def _gru(x, h, Wih, Whh, bih, bhh):
    gi = _dot(x, Wih) + bih
    gh = _dot(h, Whh) + bhh
    i_r, i_z, i_n = gi[:, :H], gi[:, H:2 * H], gi[:, 2 * H:]
    h_r, h_z, h_n = gh[:, :H], gh[:, H:2 * H], gh[:, 2 * H:]
    r = jax.nn.sigmoid(i_r + h_r)
    z = jax.nn.sigmoid(i_z + h_z)
    ncand = jnp.tanh(i_n + r * h_n)
    return (1.0 - z) * ncand + z * h


def _tc_prep_body(na, w1, b1, gwa, gw2, attr8, uy_o, rt_o, x0_o):
    x0 = _leaky(na[...] * w1[...] + b1[...])
    uy_o[:, :H] = _dot(x0, gwa[...])
    uy_o[:, H:] = _dot(x0, gw2[...])
    rt_o[...] = _dot(x0, attr8[...])
    x0_o[...] = x0


def _tc_prep(na, w1, b1, gwa, gw2, attr8):
    return pl.pallas_call(
        _tc_prep_body,
        out_shape=(jax.ShapeDtypeStruct((N, 2 * H), _f32),
                   jax.ShapeDtypeStruct((N, 8), _f32),
                   jax.ShapeDtypeStruct((N, H), _f32)),
    )(na, w1, b1, gwa, gw2, attr8)


BN = 2000


def _stage_common(acc, bias, x, Wih, Whh, bih, bhh):
    num = acc[0, :, :H] + acc[1, :, :H]
    den = acc[0, :, H:H + 1] + acc[1, :, H:H + 1]
    h = _elu(num / (den + 1e-16) + bias)
    return jax.nn.relu(_gru(h, x, Wih, Whh, bih, bhh))


def _full(ndim):
    return pl.BlockSpec(index_map=lambda i: (0,) * ndim)


def _tc_stage_body(acc, bias, x, Wih, Whh, bih, bhh, Wn, attse, attd8,
                   xn_o, xle_o, dt_o):
    xn = _stage_common(acc[...], bias[...], x[...], Wih[...], Whh[...],
                       bih[...], bhh[...])
    xl = _dot(xn, Wn[...])
    xn_o[...] = xn
    xle_o[:, :H] = xl
    xle_o[:, H:] = _dot(xl, attse[...])
    dt_o[...] = _dot(xl, attd8[...])


def _tc_stage(acc, bias, x, Wih, Whh, bih, bhh, Wn, attse, attd8):
    return pl.pallas_call(
        _tc_stage_body,
        grid=(N // BN,),
        in_specs=[
            pl.BlockSpec((NC, BN, ROWW), lambda i: (0, i, 0)),
            _full(2),
            pl.BlockSpec((BN, H), lambda i: (i, 0)),
            _full(2), _full(2), _full(2), _full(2), _full(2), _full(2),
            _full(2),
        ],
        out_specs=(pl.BlockSpec((BN, H), lambda i: (i, 0)),
                   pl.BlockSpec((BN, ROWW), lambda i: (i, 0)),
                   pl.BlockSpec((BN, 8), lambda i: (i, 0))),
        out_shape=(jax.ShapeDtypeStruct((N, H), _f32),
                   jax.ShapeDtypeStruct((N, ROWW), _f32),
                   jax.ShapeDtypeStruct((N, 8), _f32)),
    )(acc, bias, x, Wih, Whh, bih, bhh, Wn, attse, attd8)


def _tc_stage_last_body(acc, bias, x, Wih, Whh, bih, bhh, xn_o):
    xn_o[...] = _stage_common(acc[...], bias[...], x[...], Wih[...], Whh[...],
                              bih[...], bhh[...])


def _tc_stage_last(acc, bias, x, Wih, Whh, bih, bhh):
    return pl.pallas_call(
        _tc_stage_last_body,
        grid=(N // BN,),
        in_specs=[
            pl.BlockSpec((NC, BN, ROWW), lambda i: (0, i, 0)),
            _full(2),
            pl.BlockSpec((BN, H), lambda i: (i, 0)),
            _full(2), _full(2), _full(2), _full(2),
        ],
        out_specs=pl.BlockSpec((BN, H), lambda i: (i, 0)),
        out_shape=jax.ShapeDtypeStruct((N, H), _f32),
    )(acc, bias, x, Wih, Whh, bih, bhh)


def _tc_readout_body(x, mw, masrc, madst, mbias, mWih, mWhh, mbih, mbhh,
                     w2, b2, out_o):
    xv = x[...]
    out0 = jax.nn.relu(jnp.sum(xv, axis=0, keepdims=True))
    xs = _dot(xv, mw[...])
    od = _dot(out0, mw[...])
    c = _dot(od, madst[...])
    a = _leaky(_dot(xs, masrc[...]) + c)
    amax = jnp.max(a)
    ex = jnp.exp(a - amax)
    denom = jnp.sum(ex)
    p = ex / (denom + 1e-16)
    hm = _elu(jnp.sum(xs * p, axis=0, keepdims=True) + mbias[...])
    out = jax.nn.relu(_gru(hm, out0, mWih[...], mWhh[...], mbih[...],
                           mbhh[...]))
    out_o[...] = _dot(out, w2[...]) + b2[...]


def _tc_readout(x, mw, masrc, madst, mbias, mWih, mWhh, mbih, mbhh, w2, b2):
    return pl.pallas_call(
        _tc_readout_body,
        out_shape=jax.ShapeDtypeStruct((1, 1), _f32),
    )(x, mw, masrc, madst, mbias, mWih, mWhh, mbih, mbhh, w2, b2)


def _pad_cols(v, w):
    return jnp.pad(v[:, None], ((0, 0), (0, w - 1)))


def kernel(node_attr, edge_index, edge_attr, lin1_W, lin1_b, gate_lin1_W,
           gate_att_l, gate_att_r, gate_lin2_W, gate_bias, gru0_Wih,
           gru0_Whh, gru0_bih, gru0_bhh, atom_lin_W, atom_att_src,
           atom_att_dst, atom_bias, atom_gru_Wih, atom_gru_Whh, atom_gru_bih,
           atom_gru_bhh, mol_lin_W, mol_att_src, mol_att_dst, mol_bias,
           mol_gru_Wih, mol_gru_Whh, mol_gru_bih, mol_gru_bhh, lin2_W,
           lin2_b):
    pad = E_PAD - E
    src_f = jnp.concatenate([edge_index[0], jnp.zeros((pad,), _i32)])
    dst_f = jnp.concatenate([edge_index[1], jnp.zeros((pad,), _i32)])
    ea_f = jnp.concatenate([edge_attr[:, 0], jnp.zeros((pad,), _f32)])
    srcA = src_f.reshape(NT, NCHA, CA)
    dstA = dst_f.reshape(NT, NCHA, CA)
    srcG = src_f.reshape(NT, NCHG, CG)
    dstG = dst_f.reshape(NT, NCHG, CG)
    eaG = ea_f.reshape(NT, NCHG, CG)

    uy, rt, x0 = _tc_prep(node_attr, lin1_W, lin1_b.reshape(1, H),
                          gate_lin1_W[:H], gate_lin2_W,
                          _pad_cols(gate_att_r, 8))

    acc = _sc_gate(uy, rt, gate_lin1_W[H], gate_att_l, srcG, dstG, eaG)

    x, xle, dt = _tc_stage(acc, gate_bias.reshape(1, H), x0,
                           gru0_Wih, gru0_Whh, gru0_bih.reshape(1, 3 * H),
                           gru0_bhh.reshape(1, 3 * H), atom_lin_W[0],
                           _pad_cols(atom_att_src[0], 16),
                           _pad_cols(atom_att_dst[0], 8))

    for l in range(L):
        acc = _sc_gat(xle, dt, srcA, dstA)
        bias = atom_bias[l].reshape(1, H)
        bih = atom_gru_bih[l].reshape(1, 3 * H)
        bhh = atom_gru_bhh[l].reshape(1, 3 * H)
        if l < L - 1:
            x, xle, dt = _tc_stage(acc, bias, x, atom_gru_Wih[l],
                                   atom_gru_Whh[l], bih, bhh,
                                   atom_lin_W[l + 1],
                                   _pad_cols(atom_att_src[l + 1], 16),
                                   _pad_cols(atom_att_dst[l + 1], 8))
        else:
            x = _tc_stage_last(acc, bias, x, atom_gru_Wih[l],
                               atom_gru_Whh[l], bih, bhh)

    return _tc_readout(x, mol_lin_W, mol_att_src[:, None],
                       mol_att_dst[:, None], mol_bias.reshape(1, H),
                       mol_gru_Wih, mol_gru_Whh, mol_gru_bih.reshape(1, 3 * H),
                       mol_gru_bhh.reshape(1, 3 * H), lin2_W,
                       lin2_b.reshape(1, 1))


# speedup vs baseline: 22.1887x; 1.3987x over previous
import functools

import jax
import jax.numpy as jnp
from jax import lax
from jax.experimental import pallas as pl
from jax.experimental.pallas import tpu as pltpu
from jax.experimental.pallas import tpu_sc as plsc

N = 10000
E = 320000
H = 128
L = 5
NC = 2
NS = 16
NT = NC * NS
EPT = E // NT + (-(E // NT) % 64)
E_PAD = NT * EPT
ROWW = 144
NP = 10112
RPT = NP // NS

CA = 64
NCHA = EPT // CA
CG = 32
NCHG = EPT // CG

_f32 = jnp.float32
_i32 = jnp.int32


def _leaky(x):
    return jnp.where(x >= 0, x, 0.01 * x)


def _elu(x):
    return jnp.where(x > 0, x, jnp.exp(x) - 1.0)


def _zero_rows(buf, rows):
    zf = jnp.full((16,), 0.0, _f32)

    def _zrow(i, c2):
        for cc in range(ROWW // 16):
            buf[i, pl.ds(cc * 16, 16)] = zf
        return c2
    lax.fori_loop(0, rows, _zrow, 0)


def _zero_acc_slice(sbuf, accs, base, c):
    nfull = RPT // c
    for k in range(nfull):
        pltpu.sync_copy(sbuf, accs.at[pl.ds(base + k * c, c)])
    rem = RPT - nfull * c
    if rem:
        pltpu.sync_copy(sbuf.at[pl.ds(0, rem)],
                        accs.at[pl.ds(base + nfull * c, rem)])


def _sc_gat_body(xle_h, d_h, src_h, dst_h, acc_out,
                 dstv, sidx0, sidx1, gbuf0, gbuf1, dbuf0, dbuf1, sbuf, accs,
                 gsem0, gsem1, dsem0, dsem1, isem0, isem1, ssem):
    cid = lax.axis_index("c")
    sid = lax.axis_index("s")
    wid = sid * NC + cid
    lane = lax.iota(_i32, 16)
    zeros16 = jnp.full((16,), 0, _i32)
    colH16 = jnp.full((16,), H, _i32)
    sidxs = (sidx0, sidx1)
    gbufs = (gbuf0, gbuf1)
    dbufs = (dbuf0, dbuf1)
    gsems = (gsem0, gsem1)
    dsems = (dsem0, dsem1)
    isems = (isem0, isem1)

    pltpu.sync_copy(dst_h.at[wid], dstv)

    _zero_rows(sbuf, CA)
    base = sid * RPT
    _zero_acc_slice(sbuf, accs, base, CA)
    plsc.subcore_barrier()

    def issue_idx(g, s):
        pltpu.async_copy(src_h.at[wid, g], sidxs[s], isems[s])

    def wait_idx(g, s):
        pltpu.make_async_copy(src_h.at[wid, g], sidxs[s], isems[s]).wait()

    def issue_g(g, s):
        pltpu.async_copy(xle_h.at[sidxs[s]], gbufs[s], gsems[s])
        pltpu.async_copy(d_h.at[dstv.at[g]], dbufs[s], dsems[s])

    def wait_g(g, s):
        pltpu.make_async_copy(xle_h.at[sidxs[s]], gbufs[s], gsems[s]).wait()
        pltpu.make_async_copy(d_h.at[dstv.at[g]], dbufs[s], dsems[s]).wait()

    def _compute(g, s):
        gb = gbufs[s]
        db = dbufs[s]

        def _grp(j, c2):
            idx16 = j * 16 + lane
            s16 = plsc.load_gather(gb, [idx16, colH16])
            d16 = plsc.load_gather(db, [idx16, zeros16])
            a = _leaky(s16 + d16)
            eid = wid * EPT + g * CA + idx16
            ex16 = jnp.where(eid < E, jnp.exp(a), 0.0)
            for k in range(16):
                i = j * 16 + k
                ex = ex16[k]
                for cc in range(H // 16):
                    sbuf[i, pl.ds(cc * 16, 16)] = (
                        gb[i, pl.ds(cc * 16, 16)] * ex)
                sbuf[i, pl.ds(H, 16)] = jnp.where(lane == 0, ex, 0.0)
            return c2
        lax.fori_loop(0, CA // 16, _grp, 0)

    def _chunk(g, s, last=False, last2=False):
        s1 = 1 - s
        if not last:
            wait_idx(g + 1, s1)
            issue_g(g + 1, s1)
        wait_g(g, s)
        if not (last or last2):
            issue_idx(g + 2, s)
        _compute(g, s)
        pltpu.async_copy(sbuf, accs.at[dstv.at[g]], ssem, add=True).wait()

    pltpu.sync_copy(src_h.at[wid, 0], sidx0)
    issue_g(0, 0)
    issue_idx(1, 1)

    _chunk(0, 0)
    _chunk(1, 1)

    def _pair(p, carry):
        _chunk(2 * p, 0)
        _chunk(2 * p + 1, 1)
        return carry
    lax.fori_loop(1, NCHA // 2 - 1, _pair, 0)

    _chunk(NCHA - 2, 0, last2=True)
    _chunk(NCHA - 1, 1, last=True)

    plsc.subcore_barrier()
    pltpu.sync_copy(accs.at[pl.ds(base, RPT)],
                    acc_out.at[cid, pl.ds(base, RPT)])


@functools.lru_cache(maxsize=None)
def _sc_gat_kernel():
    return pl.kernel(
        _sc_gat_body,
        out_type=jax.ShapeDtypeStruct((NC, NP, ROWW), _f32),
        mesh=plsc.VectorSubcoreMesh(core_axis_name="c", subcore_axis_name="s"),
        compiler_params=pltpu.CompilerParams(use_tc_tiling_on_sc=False,
                                             needs_layout_passes=False),
        scratch_types=[
            pltpu.VMEM((NCHA, CA), _i32),
            pltpu.VMEM((CA,), _i32),
            pltpu.VMEM((CA,), _i32),
            pltpu.VMEM((CA, ROWW), _f32),
            pltpu.VMEM((CA, ROWW), _f32),
            pltpu.VMEM((CA, 8), _f32),
            pltpu.VMEM((CA, 8), _f32),
            pltpu.VMEM((CA, ROWW), _f32),
            pltpu.VMEM_SHARED((NP, ROWW), _f32),
            pltpu.SemaphoreType.DMA,
            pltpu.SemaphoreType.DMA,
            pltpu.SemaphoreType.DMA,
            pltpu.SemaphoreType.DMA,
            pltpu.SemaphoreType.DMA,
            pltpu.SemaphoreType.DMA,
            pltpu.SemaphoreType.DMA,
        ],
    )


def _sc_gat(*args):
    return _sc_gat_kernel()(*args)


def _sc_gate_body(uy_h, r_h, wrow_h, attl_h, src_h, dst_h, ea_h, acc_out,
                  dstv, sidx0, sidx1, eav0, eav1, wv, lv,
                  gbuf0, gbuf1, rbuf0, rbuf1, sbuf, accs,
                  gsem0, gsem1, rsem0, rsem1, isem0, isem1, ssem):
    cid = lax.axis_index("c")
    sid = lax.axis_index("s")
    wid = sid * NC + cid
    lane = lax.iota(_i32, 16)
    zeros16 = jnp.full((16,), 0, _i32)
    sidxs = (sidx0, sidx1)
    eavs = (eav0, eav1)
    gbufs = (gbuf0, gbuf1)
    rbufs = (rbuf0, rbuf1)
    gsems = (gsem0, gsem1)
    rsems = (rsem0, rsem1)
    isems = (isem0, isem1)

    pltpu.sync_copy(dst_h.at[wid], dstv)
    pltpu.sync_copy(wrow_h, wv)
    pltpu.sync_copy(attl_h, lv)

    _zero_rows(sbuf, CG)
    base = sid * RPT
    _zero_acc_slice(sbuf, accs, base, CG)
    plsc.subcore_barrier()

    def issue_idx(g, s):
        pltpu.async_copy(src_h.at[wid, g], sidxs[s], isems[s])
        pltpu.async_copy(ea_h.at[wid, g], eavs[s], isems[s])

    def wait_idx(g, s):
        pltpu.make_async_copy(src_h.at[wid, g], sidxs[s], isems[s]).wait()
        pltpu.make_async_copy(ea_h.at[wid, g], eavs[s], isems[s]).wait()

    def issue_g(g, s):
        pltpu.async_copy(uy_h.at[sidxs[s]], gbufs[s], gsems[s])
        pltpu.async_copy(r_h.at[dstv.at[g]], rbufs[s], rsems[s])

    def wait_g(g, s):
        pltpu.make_async_copy(uy_h.at[sidxs[s]], gbufs[s], gsems[s]).wait()
        pltpu.make_async_copy(r_h.at[dstv.at[g]], rbufs[s], rsems[s]).wait()

    def _compute(g, s):
        gb = gbufs[s]
        rb = rbufs[s]
        eav = eavs[s]

        def _grp(j, c2):
            idx16 = j * 16 + lane
            r16 = plsc.load_gather(rb, [idx16, zeros16])
            ea16 = eav[pl.ds(j * 16, 16)]
            eid = wid * EPT + g * CG + idx16
            valid16 = jnp.where(eid < E, 1.0, 0.0)
            tv = jnp.full((16,), 0.0, _f32)
            for k in range(16):
                i = j * 16 + k
                e = ea16[k]
                t16 = jnp.full((16,), 0.0, _f32)
                for cc in range(H // 16):
                    z = (gb[i, pl.ds(cc * 16, 16)]
                         + e * wv[pl.ds(cc * 16, 16)])
                    t16 = t16 + _leaky(z) * lv[pl.ds(cc * 16, 16)]
                tv = jnp.where(lane == k, jnp.sum(t16), tv)
            ex16 = jnp.exp(_leaky(tv + r16)) * valid16
            for k in range(16):
                i = j * 16 + k
                ex = ex16[k]
                for cc in range(H // 16):
                    sbuf[i, pl.ds(cc * 16, 16)] = (
                        gb[i, pl.ds(H + cc * 16, 16)] * ex)
                sbuf[i, pl.ds(H, 16)] = jnp.where(lane == 0, ex, 0.0)
            return c2
        lax.fori_loop(0, CG // 16, _grp, 0)

    def _chunk(g, s, last=False, last2=False):
        s1 = 1 - s
        if not last:
            wait_idx(g + 1, s1)
            issue_g(g + 1, s1)
        wait_g(g, s)
        _compute(g, s)
        if not (last or last2):
            issue_idx(g + 2, s)
        pltpu.async_copy(sbuf, accs.at[dstv.at[g]], ssem, add=True).wait()

    pltpu.sync_copy(src_h.at[wid, 0], sidx0)
    pltpu.sync_copy(ea_h.at[wid, 0], eav0)
    issue_g(0, 0)
    issue_idx(1, 1)

    _chunk(0, 0)
    _chunk(1, 1)

    def _pair(p, carry):
        _chunk(2 * p, 0)
        _chunk(2 * p + 1, 1)
        return carry
    lax.fori_loop(1, NCHG // 2 - 1, _pair, 0)

    _chunk(NCHG - 2, 0, last2=True)
    _chunk(NCHG - 1, 1, last=True)

    plsc.subcore_barrier()
    pltpu.sync_copy(accs.at[pl.ds(base, RPT)],
                    acc_out.at[cid, pl.ds(base, RPT)])


@functools.lru_cache(maxsize=None)
def _sc_gate_kernel():
    return pl.kernel(
        _sc_gate_body,
        out_type=jax.ShapeDtypeStruct((NC, NP, ROWW), _f32),
        mesh=plsc.VectorSubcoreMesh(core_axis_name="c", subcore_axis_name="s"),
        compiler_params=pltpu.CompilerParams(use_tc_tiling_on_sc=False,
                                             needs_layout_passes=False),
        scratch_types=[
            pltpu.VMEM((NCHG, CG), _i32),
            pltpu.VMEM((CG,), _i32),
            pltpu.VMEM((CG,), _i32),
            pltpu.VMEM((CG,), _f32),
            pltpu.VMEM((CG,), _f32),
            pltpu.VMEM((H,), _f32),
            pltpu.VMEM((H,), _f32),
            pltpu.VMEM((CG, 2 * H), _f32),
            pltpu.VMEM((CG, 2 * H), _f32),
            pltpu.VMEM((CG, 8), _f32),
            pltpu.VMEM((CG, 8), _f32),
            pltpu.VMEM((CG, ROWW), _f32),
            pltpu.VMEM_SHARED((NP, ROWW), _f32),
            pltpu.SemaphoreType.DMA,
            pltpu.SemaphoreType.DMA,
            pltpu.SemaphoreType.DMA,
            pltpu.SemaphoreType.DMA,
            pltpu.SemaphoreType.DMA,
            pltpu.SemaphoreType.DMA,
            pltpu.SemaphoreType.DMA,
        ],
    )


def _sc_gate(*args):
    return _sc_gate_kernel()(*args)


def _dot(a, b):
    return jax.lax.dot_general(a, b, (((1,), (0,)), ((), ())),
                               preferred_element_type=_f32)


def _gru(x, h, Wih, Whh, bih, bhh):
    gi = _dot(x, Wih) + bih
    gh = _dot(h, Whh) + bhh
    i_r, i_z, i_n = gi[:, :H], gi[:, H:2 * H], gi[:, 2 * H:]
    h_r, h_z, h_n = gh[:, :H], gh[:, H:2 * H], gh[:, 2 * H:]
    r = jax.nn.sigmoid(i_r + h_r)
    z = jax.nn.sigmoid(i_z + h_z)
    ncand = jnp.tanh(i_n + r * h_n)
    return (1.0 - z) * ncand + z * h


def _tc_prep_body(na, w1, b1, gwa, gw2, attr8, uy_o, rt_o, x0_o):
    x0 = _leaky(na[...] * w1[...] + b1[...])
    uy_o[:, :H] = _dot(x0, gwa[...])
    uy_o[:, H:] = _dot(x0, gw2[...])
    rt_o[...] = _dot(x0, attr8[...])
    x0_o[...] = x0


def _tc_prep(na, w1, b1, gwa, gw2, attr8):
    return pl.pallas_call(
        _tc_prep_body,
        out_shape=(jax.ShapeDtypeStruct((N, 2 * H), _f32),
                   jax.ShapeDtypeStruct((N, 8), _f32),
                   jax.ShapeDtypeStruct((N, H), _f32)),
    )(na, w1, b1, gwa, gw2, attr8)


BN = 2000


def _stage_common(acc, bias, x, Wih, Whh, bih, bhh):
    num = acc[0, :, :H] + acc[1, :, :H]
    den = acc[0, :, H:H + 1] + acc[1, :, H:H + 1]
    h = _elu(num / (den + 1e-16) + bias)
    return jax.nn.relu(_gru(h, x, Wih, Whh, bih, bhh))


def _full(ndim):
    return pl.BlockSpec(index_map=lambda i: (0,) * ndim)


def _tc_stage_body(acc, bias, x, Wih, Whh, bih, bhh, Wn, attse, attd8,
                   xn_o, xle_o, dt_o):
    xn = _stage_common(acc[...], bias[...], x[...], Wih[...], Whh[...],
                       bih[...], bhh[...])
    xl = _dot(xn, Wn[...])
    xn_o[...] = xn
    xle_o[:, :H] = xl
    xle_o[:, H:] = _dot(xl, attse[...])
    dt_o[...] = _dot(xl, attd8[...])


def _tc_stage(acc, bias, x, Wih, Whh, bih, bhh, Wn, attse, attd8):
    return pl.pallas_call(
        _tc_stage_body,
        grid=(N // BN,),
        in_specs=[
            pl.BlockSpec((NC, BN, ROWW), lambda i: (0, i, 0)),
            _full(2),
            pl.BlockSpec((BN, H), lambda i: (i, 0)),
            _full(2), _full(2), _full(2), _full(2), _full(2), _full(2),
            _full(2),
        ],
        out_specs=(pl.BlockSpec((BN, H), lambda i: (i, 0)),
                   pl.BlockSpec((BN, ROWW), lambda i: (i, 0)),
                   pl.BlockSpec((BN, 8), lambda i: (i, 0))),
        out_shape=(jax.ShapeDtypeStruct((N, H), _f32),
                   jax.ShapeDtypeStruct((N, ROWW), _f32),
                   jax.ShapeDtypeStruct((N, 8), _f32)),
    )(acc, bias, x, Wih, Whh, bih, bhh, Wn, attse, attd8)


def _tc_stage_last_body(acc, bias, x, Wih, Whh, bih, bhh, xn_o):
    xn_o[...] = _stage_common(acc[...], bias[...], x[...], Wih[...], Whh[...],
                              bih[...], bhh[...])


def _tc_stage_last(acc, bias, x, Wih, Whh, bih, bhh):
    return pl.pallas_call(
        _tc_stage_last_body,
        grid=(N // BN,),
        in_specs=[
            pl.BlockSpec((NC, BN, ROWW), lambda i: (0, i, 0)),
            _full(2),
            pl.BlockSpec((BN, H), lambda i: (i, 0)),
            _full(2), _full(2), _full(2), _full(2),
        ],
        out_specs=pl.BlockSpec((BN, H), lambda i: (i, 0)),
        out_shape=jax.ShapeDtypeStruct((N, H), _f32),
    )(acc, bias, x, Wih, Whh, bih, bhh)


def _tc_readout_body(x, mw, masrc, madst, mbias, mWih, mWhh, mbih, mbhh,
                     w2, b2, out_o):
    xv = x[...]
    out0 = jax.nn.relu(jnp.sum(xv, axis=0, keepdims=True))
    xs = _dot(xv, mw[...])
    od = _dot(out0, mw[...])
    c = _dot(od, madst[...])
    a = _leaky(_dot(xs, masrc[...]) + c)
    amax = jnp.max(a)
    ex = jnp.exp(a - amax)
    denom = jnp.sum(ex)
    p = ex / (denom + 1e-16)
    hm = _elu(jnp.sum(xs * p, axis=0, keepdims=True) + mbias[...])
    out = jax.nn.relu(_gru(hm, out0, mWih[...], mWhh[...], mbih[...],
                           mbhh[...]))
    out_o[...] = _dot(out, w2[...]) + b2[...]


def _tc_readout(x, mw, masrc, madst, mbias, mWih, mWhh, mbih, mbhh, w2, b2):
    return pl.pallas_call(
        _tc_readout_body,
        out_shape=jax.ShapeDtypeStruct((1, 1), _f32),
    )(x, mw, masrc, madst, mbias, mWih, mWhh, mbih, mbhh, w2, b2)


def _pad_cols(v, w):
    return jnp.pad(v[:, None], ((0, 0), (0, w - 1)))


def kernel(node_attr, edge_index, edge_attr, lin1_W, lin1_b, gate_lin1_W,
           gate_att_l, gate_att_r, gate_lin2_W, gate_bias, gru0_Wih,
           gru0_Whh, gru0_bih, gru0_bhh, atom_lin_W, atom_att_src,
           atom_att_dst, atom_bias, atom_gru_Wih, atom_gru_Whh, atom_gru_bih,
           atom_gru_bhh, mol_lin_W, mol_att_src, mol_att_dst, mol_bias,
           mol_gru_Wih, mol_gru_Whh, mol_gru_bih, mol_gru_bhh, lin2_W,
           lin2_b):
    pad = E_PAD - E
    src_f = jnp.concatenate([edge_index[0], jnp.zeros((pad,), _i32)])
    dst_f = jnp.concatenate([edge_index[1], jnp.zeros((pad,), _i32)])
    ea_f = jnp.concatenate([edge_attr[:, 0], jnp.zeros((pad,), _f32)])
    srcA = src_f.reshape(NT, NCHA, CA)
    dstA = dst_f.reshape(NT, NCHA, CA)
    srcG = src_f.reshape(NT, NCHG, CG)
    dstG = dst_f.reshape(NT, NCHG, CG)
    eaG = ea_f.reshape(NT, NCHG, CG)

    uy, rt, x0 = _tc_prep(node_attr, lin1_W, lin1_b.reshape(1, H),
                          gate_lin1_W[:H], gate_lin2_W,
                          _pad_cols(gate_att_r, 8))

    acc = _sc_gate(uy, rt, gate_lin1_W[H], gate_att_l, srcG, dstG, eaG)

    x, xle, dt = _tc_stage(acc, gate_bias.reshape(1, H), x0,
                           gru0_Wih, gru0_Whh, gru0_bih.reshape(1, 3 * H),
                           gru0_bhh.reshape(1, 3 * H), atom_lin_W[0],
                           _pad_cols(atom_att_src[0], 16),
                           _pad_cols(atom_att_dst[0], 8))

    for l in range(L):
        acc = _sc_gat(xle, dt, srcA, dstA)
        bias = atom_bias[l].reshape(1, H)
        bih = atom_gru_bih[l].reshape(1, 3 * H)
        bhh = atom_gru_bhh[l].reshape(1, 3 * H)
        if l < L - 1:
            x, xle, dt = _tc_stage(acc, bias, x, atom_gru_Wih[l],
                                   atom_gru_Whh[l], bih, bhh,
                                   atom_lin_W[l + 1],
                                   _pad_cols(atom_att_src[l + 1], 16),
                                   _pad_cols(atom_att_dst[l + 1], 8))
        else:
            x = _tc_stage_last(acc, bias, x, atom_gru_Wih[l],
                               atom_gru_Whh[l], bih, bhh)

    return _tc_readout(x, mol_lin_W, mol_att_src[:, None],
                       mol_att_dst[:, None], mol_bias.reshape(1, H),
                       mol_gru_Wih, mol_gru_Whh, mol_gru_bih.reshape(1, 3 * H),
                       mol_gru_bhh.reshape(1, 3 * H), lin2_W,
                       lin2_b.reshape(1, 1))


# speedup vs baseline: 22.2277x; 1.0018x over previous
import functools

import jax
import jax.numpy as jnp
from jax import lax
from jax.experimental import pallas as pl
from jax.experimental.pallas import tpu as pltpu
from jax.experimental.pallas import tpu_sc as plsc

N = 10000
E = 320000
H = 128
L = 5
NC = 2
NS = 16
NT = NC * NS
EPT = E // NT + (-(E // NT) % 64)
E_PAD = NT * EPT
ROWW = 144
NP = 10112
RPT = NP // NS

CA = 64
NCHA = EPT // CA
CG = 32
NCHG = EPT // CG

_f32 = jnp.float32
_i32 = jnp.int32


def _leaky(x):
    return jnp.where(x >= 0, x, 0.01 * x)


def _elu(x):
    return jnp.where(x > 0, x, jnp.exp(x) - 1.0)


def _zero_rows(buf, rows):
    zf = jnp.full((16,), 0.0, _f32)

    def _zrow(i, c2):
        for cc in range(ROWW // 16):
            buf[i, pl.ds(cc * 16, 16)] = zf
        return c2
    lax.fori_loop(0, rows, _zrow, 0)


def _zero_acc_slice(sbuf, accs, base, c):
    nfull = RPT // c
    for k in range(nfull):
        pltpu.sync_copy(sbuf, accs.at[pl.ds(base + k * c, c)])
    rem = RPT - nfull * c
    if rem:
        pltpu.sync_copy(sbuf.at[pl.ds(0, rem)],
                        accs.at[pl.ds(base + nfull * c, rem)])


def _sc_gat_body(xle_h, d_h, src_h, dst_h, acc_out,
                 dstv, sidx0, sidx1, gbuf0, gbuf1, dbuf0, dbuf1, sbuf, accs,
                 gsem0, gsem1, dsem0, dsem1, isem0, isem1, ssem):
    cid = lax.axis_index("c")
    sid = lax.axis_index("s")
    wid = sid * NC + cid
    lane = lax.iota(_i32, 16)
    zeros16 = jnp.full((16,), 0, _i32)
    colH16 = jnp.full((16,), H, _i32)
    sidxs = (sidx0, sidx1)
    gbufs = (gbuf0, gbuf1)
    dbufs = (dbuf0, dbuf1)
    gsems = (gsem0, gsem1)
    dsems = (dsem0, dsem1)
    isems = (isem0, isem1)

    pltpu.sync_copy(dst_h.at[wid], dstv)

    def issue_idx(g, s):
        pltpu.async_copy(src_h.at[wid, g], sidxs[s], isems[s])

    def wait_idx(g, s):
        pltpu.make_async_copy(src_h.at[wid, g], sidxs[s], isems[s]).wait()

    def issue_g(g, s):
        pltpu.async_copy(xle_h.at[sidxs[s]], gbufs[s], gsems[s])
        pltpu.async_copy(d_h.at[dstv.at[g]], dbufs[s], dsems[s])

    def wait_g(g, s):
        pltpu.make_async_copy(xle_h.at[sidxs[s]], gbufs[s], gsems[s]).wait()
        pltpu.make_async_copy(d_h.at[dstv.at[g]], dbufs[s], dsems[s]).wait()

    def _compute(g, s):
        gb = gbufs[s]
        db = dbufs[s]

        def _grp(j, c2):
            idx16 = j * 16 + lane
            s16 = plsc.load_gather(gb, [idx16, colH16])
            d16 = plsc.load_gather(db, [idx16, zeros16])
            a = _leaky(s16 + d16)
            eid = wid * EPT + g * CA + idx16
            ex16 = jnp.where(eid < E, jnp.exp(a), 0.0)
            for k in range(16):
                i = j * 16 + k
                ex = ex16[k]
                for cc in range(H // 16):
                    sbuf[i, pl.ds(cc * 16, 16)] = (
                        gb[i, pl.ds(cc * 16, 16)] * ex)
                sbuf[i, pl.ds(H, 16)] = jnp.where(lane == 0, ex, 0.0)
            return c2
        lax.fori_loop(0, CA // 16, _grp, 0)

    def _chunk(g, s, last=False, last2=False):
        s1 = 1 - s
        if not last:
            wait_idx(g + 1, s1)
            issue_g(g + 1, s1)
        wait_g(g, s)
        if not (last or last2):
            issue_idx(g + 2, s)
        _compute(g, s)
        pltpu.async_copy(sbuf, accs.at[dstv.at[g]], ssem, add=True).wait()

    pltpu.sync_copy(src_h.at[wid, 0], sidx0)
    issue_g(0, 0)
    issue_idx(1, 1)

    _zero_rows(sbuf, CA)
    base = sid * RPT
    _zero_acc_slice(sbuf, accs, base, CA)
    plsc.subcore_barrier()

    _chunk(0, 0)
    _chunk(1, 1)

    def _pair(p, carry):
        _chunk(2 * p, 0)
        _chunk(2 * p + 1, 1)
        return carry
    lax.fori_loop(1, NCHA // 2 - 1, _pair, 0)

    _chunk(NCHA - 2, 0, last2=True)
    _chunk(NCHA - 1, 1, last=True)

    plsc.subcore_barrier()
    pltpu.sync_copy(accs.at[pl.ds(base, RPT)],
                    acc_out.at[cid, pl.ds(base, RPT)])


@functools.lru_cache(maxsize=None)
def _sc_gat_kernel():
    return pl.kernel(
        _sc_gat_body,
        out_type=jax.ShapeDtypeStruct((NC, NP, ROWW), _f32),
        mesh=plsc.VectorSubcoreMesh(core_axis_name="c", subcore_axis_name="s"),
        compiler_params=pltpu.CompilerParams(use_tc_tiling_on_sc=False,
                                             needs_layout_passes=False),
        scratch_types=[
            pltpu.VMEM((NCHA, CA), _i32),
            pltpu.VMEM((CA,), _i32),
            pltpu.VMEM((CA,), _i32),
            pltpu.VMEM((CA, ROWW), _f32),
            pltpu.VMEM((CA, ROWW), _f32),
            pltpu.VMEM((CA, 8), _f32),
            pltpu.VMEM((CA, 8), _f32),
            pltpu.VMEM((CA, ROWW), _f32),
            pltpu.VMEM_SHARED((NP, ROWW), _f32),
            pltpu.SemaphoreType.DMA,
            pltpu.SemaphoreType.DMA,
            pltpu.SemaphoreType.DMA,
            pltpu.SemaphoreType.DMA,
            pltpu.SemaphoreType.DMA,
            pltpu.SemaphoreType.DMA,
            pltpu.SemaphoreType.DMA,
        ],
    )


def _sc_gat(*args):
    return _sc_gat_kernel()(*args)


def _sc_gate_body(uy_h, r_h, wrow_h, attl_h, src_h, dst_h, ea_h, acc_out,
                  dstv, sidx0, sidx1, eav0, eav1, wv, lv,
                  gbuf0, gbuf1, rbuf0, rbuf1, sbuf, accs,
                  gsem0, gsem1, rsem0, rsem1, isem0, isem1, ssem):
    cid = lax.axis_index("c")
    sid = lax.axis_index("s")
    wid = sid * NC + cid
    lane = lax.iota(_i32, 16)
    zeros16 = jnp.full((16,), 0, _i32)
    sidxs = (sidx0, sidx1)
    eavs = (eav0, eav1)
    gbufs = (gbuf0, gbuf1)
    rbufs = (rbuf0, rbuf1)
    gsems = (gsem0, gsem1)
    rsems = (rsem0, rsem1)
    isems = (isem0, isem1)

    pltpu.sync_copy(dst_h.at[wid], dstv)
    pltpu.sync_copy(wrow_h, wv)
    pltpu.sync_copy(attl_h, lv)

    def issue_idx(g, s):
        pltpu.async_copy(src_h.at[wid, g], sidxs[s], isems[s])
        pltpu.async_copy(ea_h.at[wid, g], eavs[s], isems[s])

    def wait_idx(g, s):
        pltpu.make_async_copy(src_h.at[wid, g], sidxs[s], isems[s]).wait()
        pltpu.make_async_copy(ea_h.at[wid, g], eavs[s], isems[s]).wait()

    def issue_g(g, s):
        pltpu.async_copy(uy_h.at[sidxs[s]], gbufs[s], gsems[s])
        pltpu.async_copy(r_h.at[dstv.at[g]], rbufs[s], rsems[s])

    def wait_g(g, s):
        pltpu.make_async_copy(uy_h.at[sidxs[s]], gbufs[s], gsems[s]).wait()
        pltpu.make_async_copy(r_h.at[dstv.at[g]], rbufs[s], rsems[s]).wait()

    def _compute(g, s):
        gb = gbufs[s]
        rb = rbufs[s]
        eav = eavs[s]

        def _grp(j, c2):
            idx16 = j * 16 + lane
            r16 = plsc.load_gather(rb, [idx16, zeros16])
            ea16 = eav[pl.ds(j * 16, 16)]
            eid = wid * EPT + g * CG + idx16
            valid16 = jnp.where(eid < E, 1.0, 0.0)
            tv = jnp.full((16,), 0.0, _f32)
            for k in range(16):
                i = j * 16 + k
                e = ea16[k]
                t16 = jnp.full((16,), 0.0, _f32)
                for cc in range(H // 16):
                    z = (gb[i, pl.ds(cc * 16, 16)]
                         + e * wv[pl.ds(cc * 16, 16)])
                    t16 = t16 + _leaky(z) * lv[pl.ds(cc * 16, 16)]
                tv = jnp.where(lane == k, jnp.sum(t16), tv)
            ex16 = jnp.exp(_leaky(tv + r16)) * valid16
            for k in range(16):
                i = j * 16 + k
                ex = ex16[k]
                for cc in range(H // 16):
                    sbuf[i, pl.ds(cc * 16, 16)] = (
                        gb[i, pl.ds(H + cc * 16, 16)] * ex)
                sbuf[i, pl.ds(H, 16)] = jnp.where(lane == 0, ex, 0.0)
            return c2
        lax.fori_loop(0, CG // 16, _grp, 0)

    def _chunk(g, s, last=False, last2=False):
        s1 = 1 - s
        if not last:
            wait_idx(g + 1, s1)
            issue_g(g + 1, s1)
        wait_g(g, s)
        _compute(g, s)
        if not (last or last2):
            issue_idx(g + 2, s)
        pltpu.async_copy(sbuf, accs.at[dstv.at[g]], ssem, add=True).wait()

    pltpu.sync_copy(src_h.at[wid, 0], sidx0)
    pltpu.sync_copy(ea_h.at[wid, 0], eav0)
    issue_g(0, 0)
    issue_idx(1, 1)

    _zero_rows(sbuf, CG)
    base = sid * RPT
    _zero_acc_slice(sbuf, accs, base, CG)
    plsc.subcore_barrier()

    _chunk(0, 0)
    _chunk(1, 1)

    def _pair(p, carry):
        _chunk(2 * p, 0)
        _chunk(2 * p + 1, 1)
        return carry
    lax.fori_loop(1, NCHG // 2 - 1, _pair, 0)

    _chunk(NCHG - 2, 0, last2=True)
    _chunk(NCHG - 1, 1, last=True)

    plsc.subcore_barrier()
    pltpu.sync_copy(accs.at[pl.ds(base, RPT)],
                    acc_out.at[cid, pl.ds(base, RPT)])


@functools.lru_cache(maxsize=None)
def _sc_gate_kernel():
    return pl.kernel(
        _sc_gate_body,
        out_type=jax.ShapeDtypeStruct((NC, NP, ROWW), _f32),
        mesh=plsc.VectorSubcoreMesh(core_axis_name="c", subcore_axis_name="s"),
        compiler_params=pltpu.CompilerParams(use_tc_tiling_on_sc=False,
                                             needs_layout_passes=False),
        scratch_types=[
            pltpu.VMEM((NCHG, CG), _i32),
            pltpu.VMEM((CG,), _i32),
            pltpu.VMEM((CG,), _i32),
            pltpu.VMEM((CG,), _f32),
            pltpu.VMEM((CG,), _f32),
            pltpu.VMEM((H,), _f32),
            pltpu.VMEM((H,), _f32),
            pltpu.VMEM((CG, 2 * H), _f32),
            pltpu.VMEM((CG, 2 * H), _f32),
            pltpu.VMEM((CG, 8), _f32),
            pltpu.VMEM((CG, 8), _f32),
            pltpu.VMEM((CG, ROWW), _f32),
            pltpu.VMEM_SHARED((NP, ROWW), _f32),
            pltpu.SemaphoreType.DMA,
            pltpu.SemaphoreType.DMA,
            pltpu.SemaphoreType.DMA,
            pltpu.SemaphoreType.DMA,
            pltpu.SemaphoreType.DMA,
            pltpu.SemaphoreType.DMA,
            pltpu.SemaphoreType.DMA,
        ],
    )


def _sc_gate(*args):
    return _sc_gate_kernel()(*args)


def _dot(a, b):
    return jax.lax.dot_general(a, b, (((1,), (0,)), ((), ())),
                               preferred_element_type=_f32)


def _gru(x, h, Wih, Whh, bih, bhh):
    gi = _dot(x, Wih) + bih
    gh = _dot(h, Whh) + bhh
    i_r, i_z, i_n = gi[:, :H], gi[:, H:2 * H], gi[:, 2 * H:]
    h_r, h_z, h_n = gh[:, :H], gh[:, H:2 * H], gh[:, 2 * H:]
    r = jax.nn.sigmoid(i_r + h_r)
    z = jax.nn.sigmoid(i_z + h_z)
    ncand = jnp.tanh(i_n + r * h_n)
    return (1.0 - z) * ncand + z * h


def _tc_prep_body(na, w1, b1, gwa, gw2, attr8, uy_o, rt_o, x0_o):
    x0 = _leaky(na[...] * w1[...] + b1[...])
    uy_o[:, :H] = _dot(x0, gwa[...])
    uy_o[:, H:] = _dot(x0, gw2[...])
    rt_o[...] = _dot(x0, attr8[...])
    x0_o[...] = x0


def _tc_prep(na, w1, b1, gwa, gw2, attr8):
    return pl.pallas_call(
        _tc_prep_body,
        out_shape=(jax.ShapeDtypeStruct((N, 2 * H), _f32),
                   jax.ShapeDtypeStruct((N, 8), _f32),
                   jax.ShapeDtypeStruct((N, H), _f32)),
    )(na, w1, b1, gwa, gw2, attr8)


BN = 2000


def _stage_common(acc, bias, x, Wih, Whh, bih, bhh):
    num = acc[0, :, :H] + acc[1, :, :H]
    den = acc[0, :, H:H + 1] + acc[1, :, H:H + 1]
    h = _elu(num / (den + 1e-16) + bias)
    return jax.nn.relu(_gru(h, x, Wih, Whh, bih, bhh))


def _full(ndim):
    return pl.BlockSpec(index_map=lambda i: (0,) * ndim)


def _tc_stage_body(acc, bias, x, Wih, Whh, bih, bhh, Wn, attse, attd8,
                   xn_o, xle_o, dt_o):
    xn = _stage_common(acc[...], bias[...], x[...], Wih[...], Whh[...],
                       bih[...], bhh[...])
    xl = _dot(xn, Wn[...])
    xn_o[...] = xn
    xle_o[:, :H] = xl
    xle_o[:, H:] = _dot(xl, attse[...])
    dt_o[...] = _dot(xl, attd8[...])


def _tc_stage(acc, bias, x, Wih, Whh, bih, bhh, Wn, attse, attd8):
    return pl.pallas_call(
        _tc_stage_body,
        grid=(N // BN,),
        in_specs=[
            pl.BlockSpec((NC, BN, ROWW), lambda i: (0, i, 0)),
            _full(2),
            pl.BlockSpec((BN, H), lambda i: (i, 0)),
            _full(2), _full(2), _full(2), _full(2), _full(2), _full(2),
            _full(2),
        ],
        out_specs=(pl.BlockSpec((BN, H), lambda i: (i, 0)),
                   pl.BlockSpec((BN, ROWW), lambda i: (i, 0)),
                   pl.BlockSpec((BN, 8), lambda i: (i, 0))),
        out_shape=(jax.ShapeDtypeStruct((N, H), _f32),
                   jax.ShapeDtypeStruct((N, ROWW), _f32),
                   jax.ShapeDtypeStruct((N, 8), _f32)),
    )(acc, bias, x, Wih, Whh, bih, bhh, Wn, attse, attd8)


def _tc_stage_last_body(acc, bias, x, Wih, Whh, bih, bhh, xn_o):
    xn_o[...] = _stage_common(acc[...], bias[...], x[...], Wih[...], Whh[...],
                              bih[...], bhh[...])


def _tc_stage_last(acc, bias, x, Wih, Whh, bih, bhh):
    return pl.pallas_call(
        _tc_stage_last_body,
        grid=(N // BN,),
        in_specs=[
            pl.BlockSpec((NC, BN, ROWW), lambda i: (0, i, 0)),
            _full(2),
            pl.BlockSpec((BN, H), lambda i: (i, 0)),
            _full(2), _full(2), _full(2), _full(2),
        ],
        out_specs=pl.BlockSpec((BN, H), lambda i: (i, 0)),
        out_shape=jax.ShapeDtypeStruct((N, H), _f32),
    )(acc, bias, x, Wih, Whh, bih, bhh)


def _tc_readout_body(x, mw, masrc, madst, mbias, mWih, mWhh, mbih, mbhh,
                     w2, b2, out_o):
    xv = x[...]
    out0 = jax.nn.relu(jnp.sum(xv, axis=0, keepdims=True))
    xs = _dot(xv, mw[...])
    od = _dot(out0, mw[...])
    c = _dot(od, madst[...])
    a = _leaky(_dot(xs, masrc[...]) + c)
    amax = jnp.max(a)
    ex = jnp.exp(a - amax)
    denom = jnp.sum(ex)
    p = ex / (denom + 1e-16)
    hm = _elu(jnp.sum(xs * p, axis=0, keepdims=True) + mbias[...])
    out = jax.nn.relu(_gru(hm, out0, mWih[...], mWhh[...], mbih[...],
                           mbhh[...]))
    out_o[...] = _dot(out, w2[...]) + b2[...]


def _tc_readout(x, mw, masrc, madst, mbias, mWih, mWhh, mbih, mbhh, w2, b2):
    return pl.pallas_call(
        _tc_readout_body,
        out_shape=jax.ShapeDtypeStruct((1, 1), _f32),
    )(x, mw, masrc, madst, mbias, mWih, mWhh, mbih, mbhh, w2, b2)


def _pad_cols(v, w):
    return jnp.pad(v[:, None], ((0, 0), (0, w - 1)))


def kernel(node_attr, edge_index, edge_attr, lin1_W, lin1_b, gate_lin1_W,
           gate_att_l, gate_att_r, gate_lin2_W, gate_bias, gru0_Wih,
           gru0_Whh, gru0_bih, gru0_bhh, atom_lin_W, atom_att_src,
           atom_att_dst, atom_bias, atom_gru_Wih, atom_gru_Whh, atom_gru_bih,
           atom_gru_bhh, mol_lin_W, mol_att_src, mol_att_dst, mol_bias,
           mol_gru_Wih, mol_gru_Whh, mol_gru_bih, mol_gru_bhh, lin2_W,
           lin2_b):
    pad = E_PAD - E
    src_f = jnp.concatenate([edge_index[0], jnp.zeros((pad,), _i32)])
    dst_f = jnp.concatenate([edge_index[1], jnp.zeros((pad,), _i32)])
    ea_f = jnp.concatenate([edge_attr[:, 0], jnp.zeros((pad,), _f32)])
    srcA = src_f.reshape(NT, NCHA, CA)
    dstA = dst_f.reshape(NT, NCHA, CA)
    srcG = src_f.reshape(NT, NCHG, CG)
    dstG = dst_f.reshape(NT, NCHG, CG)
    eaG = ea_f.reshape(NT, NCHG, CG)

    uy, rt, x0 = _tc_prep(node_attr, lin1_W, lin1_b.reshape(1, H),
                          gate_lin1_W[:H], gate_lin2_W,
                          _pad_cols(gate_att_r, 8))

    acc = _sc_gate(uy, rt, gate_lin1_W[H], gate_att_l, srcG, dstG, eaG)

    x, xle, dt = _tc_stage(acc, gate_bias.reshape(1, H), x0,
                           gru0_Wih, gru0_Whh, gru0_bih.reshape(1, 3 * H),
                           gru0_bhh.reshape(1, 3 * H), atom_lin_W[0],
                           _pad_cols(atom_att_src[0], 16),
                           _pad_cols(atom_att_dst[0], 8))

    for l in range(L):
        acc = _sc_gat(xle, dt, srcA, dstA)
        bias = atom_bias[l].reshape(1, H)
        bih = atom_gru_bih[l].reshape(1, 3 * H)
        bhh = atom_gru_bhh[l].reshape(1, 3 * H)
        if l < L - 1:
            x, xle, dt = _tc_stage(acc, bias, x, atom_gru_Wih[l],
                                   atom_gru_Whh[l], bih, bhh,
                                   atom_lin_W[l + 1],
                                   _pad_cols(atom_att_src[l + 1], 16),
                                   _pad_cols(atom_att_dst[l + 1], 8))
        else:
            x = _tc_stage_last(acc, bias, x, atom_gru_Wih[l],
                               atom_gru_Whh[l], bih, bhh)

    return _tc_readout(x, mol_lin_W, mol_att_src[:, None],
                       mol_att_dst[:, None], mol_bias.reshape(1, H),
                       mol_gru_Wih, mol_gru_Whh, mol_gru_bih.reshape(1, 3 * H),
                       mol_gru_bhh.reshape(1, 3 * H), lin2_W,
                       lin2_b.reshape(1, 1))
